# Initial kernel scaffold; baseline (speedup 1.0000x reference)
import jax
import jax.numpy as jnp
from jax.experimental import pallas as pl


def kernel(x, adj, w_j, W1, W2):
    raise NotImplementedError("write your pallas kernel here")


# speedup vs baseline: 1.0868x; 1.0868x over previous
import jax
import jax.numpy as jnp
from jax.experimental import pallas as pl
from jax.experimental.pallas import tpu as pltpu


def _gnn_kernel(x_ref, adj_ref, wj_ref, w1_ref, w2_ref, out_ref, acc_ref):
    k = pl.program_id(1)
    nk = pl.num_programs(1)

    d = w1_ref.shape[0]
    xs = x_ref[0, :, pl.ds(k * d, d)]
    y = jnp.dot(xs, w1_ref[...], preferred_element_type=jnp.float32)

    a = adj_ref[0, 0].astype(jnp.bfloat16)
    m = jnp.dot(a, y.astype(jnp.bfloat16),
                preferred_element_type=jnp.float32)

    wj = wj_ref[0, 0]
    phi = jnp.maximum(m, 0.0) * wj

    @pl.when(k == 0)
    def _():
        acc_ref[...] = phi

    @pl.when(k > 0)
    def _():
        acc_ref[...] = acc_ref[...] + phi

    @pl.when(k == nk - 1)
    def _():
        out_ref[0] = jnp.dot(acc_ref[...].astype(jnp.bfloat16), w2_ref[...],
                             preferred_element_type=jnp.float32)


def kernel(x, adj, w_j, W1, W2):
    B, N, K, D = x.shape
    H = W1.shape[1]
    DOUT = W2.shape[1]

    xb = x.astype(jnp.bfloat16).reshape(B, N, K * D)
    w1b = W1.astype(jnp.bfloat16)
    w2b = W2.astype(jnp.bfloat16)
    wjt = jnp.transpose(w_j, (0, 2, 1)).astype(jnp.float32).reshape(B, K, N, 1)

    return pl.pallas_call(
        _gnn_kernel,
        grid=(B, K),
        in_specs=[
            pl.BlockSpec((1, N, K * D), lambda b, k: (b, 0, 0)),
            pl.BlockSpec((1, 1, N, N), lambda b, k: (b, k, 0, 0)),
            pl.BlockSpec((1, 1, N, 1), lambda b, k: (b, k, 0, 0)),
            pl.BlockSpec((D, H), lambda b, k: (0, 0)),
            pl.BlockSpec((H, DOUT), lambda b, k: (0, 0)),
        ],
        out_specs=pl.BlockSpec((1, N, DOUT), lambda b, k: (b, 0, 0)),
        out_shape=jax.ShapeDtypeStruct((B, N, DOUT), jnp.float32),
        scratch_shapes=[pltpu.VMEM((N, H), jnp.float32)],
    )(xb, adj, wjt, w1b, w2b)
